# baseline (speedup 1.0000x reference)
import jax
import jax.numpy as jnp
from jax.experimental import pallas as pl
from jax.experimental.pallas import tpu as pltpu

BATCH = 1024
UNITS = 2048
D_IN = 512
U_TILE = 256


def _cell_body(x_ref, ps_ref, k_ref, r_ref, b_ref, o_ref, xb_ref, pb_ref):
    @pl.when(pl.program_id(0) == 0)
    def _cache_bf16():
        xb_ref[...] = x_ref[...].astype(jnp.bfloat16)
        pb_ref[...] = ps_ref[...].astype(jnp.bfloat16)

    ip = jnp.dot(
        xb_ref[...],
        k_ref[...].astype(jnp.bfloat16),
        preferred_element_type=jnp.float32,
    )
    sp = jnp.dot(
        pb_ref[...],
        r_ref[...].astype(jnp.bfloat16),
        preferred_element_type=jnp.float32,
    )
    o_ref[...] = jnp.tanh(ip + sp + b_ref[...])


def kernel(inputs, prev_state, kernel, recurrent_kernel, bias):
    bias2 = bias.reshape(1, UNITS)
    grid = (UNITS // U_TILE,)
    out = pl.pallas_call(
        _cell_body,
        grid=grid,
        in_specs=[
            pl.BlockSpec((BATCH, D_IN), lambda u: (0, 0)),
            pl.BlockSpec((BATCH, UNITS), lambda u: (0, 0)),
            pl.BlockSpec((D_IN, U_TILE), lambda u: (0, u)),
            pl.BlockSpec((UNITS, U_TILE), lambda u: (0, u)),
            pl.BlockSpec((1, U_TILE), lambda u: (0, u)),
        ],
        out_specs=pl.BlockSpec((BATCH, U_TILE), lambda u: (0, u)),
        out_shape=jax.ShapeDtypeStruct((BATCH, UNITS), jnp.float32),
        scratch_shapes=[
            pltpu.VMEM((BATCH, D_IN), jnp.bfloat16),
            pltpu.VMEM((BATCH, UNITS), jnp.bfloat16),
        ],
        compiler_params=pltpu.CompilerParams(
            dimension_semantics=("arbitrary",),
        ),
    )(inputs, prev_state, kernel, recurrent_kernel, bias2)
    return out


# speedup vs baseline: 1.0364x; 1.0364x over previous
import jax
import jax.numpy as jnp
from jax.experimental import pallas as pl
from jax.experimental.pallas import tpu as pltpu

BATCH = 1024
UNITS = 2048
D_IN = 512
BT = 256


def _cell_body(x_ref, ps_ref, k_ref, r_ref, b_ref, o_ref):
    ip = jnp.dot(x_ref[...], k_ref[...], preferred_element_type=jnp.float32)
    sp = jnp.dot(ps_ref[...], r_ref[...], preferred_element_type=jnp.float32)
    o_ref[...] = jnp.tanh(ip + sp + b_ref[...])


def kernel(inputs, prev_state, kernel, recurrent_kernel, bias):
    bias2 = bias.reshape(1, UNITS)
    out = pl.pallas_call(
        _cell_body,
        grid=(BATCH // BT,),
        in_specs=[
            pl.BlockSpec((BT, D_IN), lambda i: (i, 0)),
            pl.BlockSpec((BT, UNITS), lambda i: (i, 0)),
            pl.BlockSpec((D_IN, UNITS), lambda i: (0, 0)),
            pl.BlockSpec((UNITS, UNITS), lambda i: (0, 0)),
            pl.BlockSpec((1, UNITS), lambda i: (0, 0)),
        ],
        out_specs=pl.BlockSpec((BT, UNITS), lambda i: (i, 0)),
        out_shape=jax.ShapeDtypeStruct((BATCH, UNITS), jnp.float32),
    )(inputs, prev_state, kernel, recurrent_kernel, bias2)
    return out
